# baseline (speedup 1.0000x reference)
import functools

import jax
import jax.numpy as jnp
from jax import lax
from jax.experimental import pallas as pl
from jax.experimental.pallas import tpu as pltpu
from jax.experimental.pallas import tpu_sc as plsc

N = 10000
FEATS = 128
HID = 64
HEADS = 8
DH = 8
CLASSES = 40
CP = 48

NC = 2
NS = 16
NW = NC * NS
L = 16

W1R = HID + L
W2R = CP + L

NP = 10240
STRIPE = NP // NS
B = 128
NCH = 84
NGB = 3
NSB = 2
C = NCH * B
E2P = NW * C

_F32 = jnp.float32
_I32 = jnp.int32


def _iota16():
    return lax.iota(_I32, L)


def _vperm(v, idx):
    dn = lax.GatherDimensionNumbers(
        offset_dims=(), collapsed_slice_dims=(0,), start_index_map=(0,))
    return lax.gather(v, idx[:, None], dn, (1,),
                      mode=lax.GatherScatterMode.PROMISE_IN_BOUNDS)


def _tc1_body(xp_ref, w1_ref, amap_s_ref, amap_d_ref,
              h1_ref, asd_ref, add_ref, m1_ref):
    h = jnp.dot(xp_ref[...], w1_ref[...], preferred_element_type=_F32)
    h1_ref[...] = h
    a_s = jnp.dot(h, amap_s_ref[...], preferred_element_type=_F32)
    a_d = jnp.dot(h, amap_d_ref[...], preferred_element_type=_F32)
    asd_ref[...] = jnp.concatenate([a_s, a_s], axis=1)
    add_ref[...] = jnp.concatenate([a_d, a_d], axis=1)
    m = jnp.maximum(jnp.max(a_s, axis=0) + jnp.max(a_d, axis=0), 0.0)
    m1_ref[...] = jnp.concatenate([m, m], axis=0)


def _tc2_body(part_ref, b1_ref, w2_ref, as2w_ref, ad2w_ref, e8_ref,
              h2p_ref, as2_ref, ad2_ref, m2_ref):
    den = part_ref[0, :, HID:HID + HEADS] + part_ref[1, :, HID:HID + HEADS]
    agg = part_ref[0, :, :HID] + part_ref[1, :, :HID]
    inv = 1.0 / (den + 1e-16)
    invx = jnp.dot(inv, e8_ref[...], preferred_element_type=_F32)
    h1f = agg * invx + b1_ref[...][None, :]
    rowid = lax.broadcasted_iota(_I32, (NP, 1), 0)
    h1f = jnp.where(rowid < N, h1f, 0.0)
    h2 = jnp.dot(h1f, w2_ref[...], preferred_element_type=_F32)
    h2p_ref[...] = jnp.pad(h2, ((0, 0), (0, CP - CLASSES)))
    a_s = jnp.dot(h2, as2w_ref[...].reshape(CLASSES, 1),
                  preferred_element_type=_F32)
    a_d = jnp.dot(h2, ad2w_ref[...].reshape(CLASSES, 1),
                  preferred_element_type=_F32)
    a_s = jnp.where(rowid < N, a_s, 0.0)
    a_d = jnp.where(rowid < N, a_d, 0.0)
    as2_ref[...] = a_s
    ad2_ref[...] = a_d
    m2 = jnp.maximum(jnp.max(a_s) + jnp.max(a_d), 0.0)
    m2_ref[...] = jnp.full((L,), m2, dtype=_F32)


def _tc3_body(part_ref, b2_ref, y_ref):
    den = part_ref[0, :, CP] + part_ref[1, :, CP]
    agg = part_ref[0, :, :CLASSES] + part_ref[1, :, :CLASSES]
    inv = 1.0 / (den + 1e-16)
    y = agg * inv[:, None] + b2_ref[...][None, :]
    y_ref[...] = y[:N, :]


def _sc1_body(src_hbm, dst_hbm, asd_hbm, add_hbm, h1_hbm, m1_hbm,
              part_out,
              sidx, didx, sbuf, dbuf, hbuf, msgb, mvec,
              gsem, ssem, out_sh):
    c = lax.axis_index("c")
    s = lax.axis_index("s")
    w = c * NS + s

    def _z(i, _):
        for q in range(W1R // L):
            msgb[0, i, pl.ds(q * L, L)] = jnp.zeros((L,), _F32)
        return 0
    lax.fori_loop(0, B, _z, 0)

    for r in range(STRIPE // B):
        row = s * STRIPE + r * B
        pltpu.sync_copy(msgb.at[0], out_sh.at[pl.ds(row, B)])
    plsc.subcore_barrier()

    pltpu.sync_copy(src_hbm.at[w], sidx)
    pltpu.sync_copy(dst_hbm.at[w], didx)
    pltpu.sync_copy(m1_hbm, mvec)

    def _gathers(ch, g):
        pltpu.async_copy(asd_hbm.at[sidx.at[ch]], sbuf.at[g], gsem.at[g])
        pltpu.async_copy(add_hbm.at[didx.at[ch]], dbuf.at[g], gsem.at[g])
        pltpu.async_copy(h1_hbm.at[sidx.at[ch]], hbuf.at[g], gsem.at[g])

    def _gwait(ch, g):
        pltpu.make_async_copy(asd_hbm.at[sidx.at[ch]], sbuf.at[g],
                              gsem.at[g]).wait()
        pltpu.make_async_copy(add_hbm.at[didx.at[ch]], dbuf.at[g],
                              gsem.at[g]).wait()
        pltpu.make_async_copy(h1_hbm.at[sidx.at[ch]], hbuf.at[g],
                              gsem.at[g]).wait()

    def _swait(ch, mb):
        pltpu.make_async_copy(msgb.at[mb], out_sh.at[didx.at[ch]],
                              ssem.at[mb]).wait()

    for g in range(NGB):
        _gathers(g, g)

    m = mvec[...]
    half = lax.shift_right_logical(_iota16(), 3)
    idxq = [half + (2 * q) for q in range(4)]

    def _super(i, _):
        for b in range(6):
            ch = 6 * i + b
            g = b % NGB
            mb = b % NSB
            _gwait(ch, g)

            @pl.when(ch >= NSB)
            def _():
                _swait(ch - NSB, mb)

            def _edge(e2, _):
                for de in range(2):
                    e = 2 * e2 + de
                    t = sbuf[g, e, :] + dbuf[g, e, :]
                    a = jnp.maximum(t, 0.2 * t)
                    ex = jnp.exp(a - m)
                    msgb[mb, e, pl.ds(HID, L)] = ex
                    for q in range(4):
                        wv = _vperm(ex, idxq[q])
                        msgb[mb, e, pl.ds(q * L, L)] = \
                            wv * hbuf[g, e, pl.ds(q * L, L)]
                return 0

            lax.fori_loop(0, B // 2, _edge, 0)
            pltpu.async_copy(msgb.at[mb], out_sh.at[didx.at[ch]],
                             ssem.at[mb], add=True)

            @pl.when(ch + NGB < NCH)
            def _():
                _gathers(ch + NGB, g)
        return 0

    lax.fori_loop(0, NCH // 6, _super, 0)
    for ch in range(NCH - NSB, NCH):
        _swait(ch, ch % NSB)
    plsc.subcore_barrier()

    row = s * STRIPE
    pltpu.sync_copy(out_sh.at[pl.ds(row, STRIPE)],
                    part_out.at[c].at[pl.ds(row, STRIPE)])


def _sc2_body(src_hbm, dst_hbm, as2_hbm, ad2_hbm, h2_hbm, m2_hbm,
              part_out,
              sidx, didx, astab, adtab, hbuf, msgb, mvec,
              gsem, ssem, out_sh):
    c = lax.axis_index("c")
    s = lax.axis_index("s")
    w = c * NS + s

    def _z(i, _):
        for q in range(W2R // L):
            msgb[0, i, pl.ds(q * L, L)] = jnp.zeros((L,), _F32)
        return 0
    lax.fori_loop(0, B, _z, 0)

    for r in range(STRIPE // B):
        row = s * STRIPE + r * B
        pltpu.sync_copy(msgb.at[0], out_sh.at[pl.ds(row, B)])
    plsc.subcore_barrier()

    pltpu.sync_copy(src_hbm.at[w], sidx)
    pltpu.sync_copy(dst_hbm.at[w], didx)
    pltpu.sync_copy(m2_hbm, mvec)
    pltpu.sync_copy(as2_hbm, astab)
    pltpu.sync_copy(ad2_hbm, adtab)

    def _gwait(ch, g):
        pltpu.make_async_copy(h2_hbm.at[sidx.at[ch]], hbuf.at[g],
                              gsem.at[g]).wait()

    def _swait(ch, mb):
        pltpu.make_async_copy(msgb.at[mb], out_sh.at[didx.at[ch]],
                              ssem.at[mb]).wait()

    for g in range(NGB):
        pltpu.async_copy(h2_hbm.at[sidx.at[g]], hbuf.at[g], gsem.at[g])

    m = mvec[...]

    def _super(i, _):
        for b in range(6):
            ch = 6 * i + b
            g = b % NGB
            mb = b % NSB
            _gwait(ch, g)

            @pl.when(ch >= NSB)
            def _():
                _swait(ch - NSB, mb)

            def _grp(gr, _):
                sv = sidx[ch, pl.ds(gr * L, L)]
                dv = didx[ch, pl.ds(gr * L, L)]
                t = (plsc.load_gather(astab, [sv]) +
                     plsc.load_gather(adtab, [dv]))
                a = jnp.maximum(t, 0.2 * t)
                exg = jnp.exp(a - m)

                def _edge(i2, _):
                    e = gr * L + i2
                    wv = _vperm(exg, jnp.full((L,), i2, dtype=_I32))
                    msgb[mb, e, pl.ds(CP, L)] = wv
                    for q in range(3):
                        msgb[mb, e, pl.ds(q * L, L)] = \
                            wv * hbuf[g, e, pl.ds(q * L, L)]
                    return 0

                lax.fori_loop(0, L, _edge, 0)
                return 0

            lax.fori_loop(0, B // L, _grp, 0)
            pltpu.async_copy(msgb.at[mb], out_sh.at[didx.at[ch]],
                             ssem.at[mb], add=True)

            @pl.when(ch + NGB < NCH)
            def _():
                pltpu.async_copy(h2_hbm.at[sidx.at[ch + NGB]], hbuf.at[g],
                                 gsem.at[g])
        return 0

    lax.fori_loop(0, NCH // 6, _super, 0)
    for ch in range(NCH - NSB, NCH):
        _swait(ch, ch % NSB)
    plsc.subcore_barrier()

    row = s * STRIPE
    pltpu.sync_copy(out_sh.at[pl.ds(row, STRIPE)],
                    part_out.at[c].at[pl.ds(row, STRIPE)])


@functools.lru_cache(maxsize=1)
def _make_kernels():
    tc1 = pl.pallas_call(
        _tc1_body,
        out_shape=[
            jax.ShapeDtypeStruct((NP, HID), _F32),
            jax.ShapeDtypeStruct((NP, 2 * HEADS), _F32),
            jax.ShapeDtypeStruct((NP, 2 * HEADS), _F32),
            jax.ShapeDtypeStruct((L,), _F32),
        ],
    )
    tc2 = pl.pallas_call(
        _tc2_body,
        out_shape=[
            jax.ShapeDtypeStruct((NP, CP), _F32),
            jax.ShapeDtypeStruct((NP, 1), _F32),
            jax.ShapeDtypeStruct((NP, 1), _F32),
            jax.ShapeDtypeStruct((L,), _F32),
        ],
    )
    tc3 = pl.pallas_call(
        _tc3_body,
        out_shape=jax.ShapeDtypeStruct((N, CLASSES), _F32),
    )
    mesh = plsc.VectorSubcoreMesh(
        core_axis_name="c", subcore_axis_name="s",
        num_cores=NC, num_subcores=NS)
    sc_params = pltpu.CompilerParams(use_tc_tiling_on_sc=False,
                                     needs_layout_passes=False)
    sc1 = pl.kernel(
        _sc1_body,
        out_type=[
            jax.ShapeDtypeStruct((NC, NP, W1R), _F32),
        ],
        mesh=mesh,
        scratch_types=[
            pltpu.VMEM((NCH, B), _I32),
            pltpu.VMEM((NCH, B), _I32),
            pltpu.VMEM((NGB, B, 2 * HEADS), _F32),
            pltpu.VMEM((NGB, B, 2 * HEADS), _F32),
            pltpu.VMEM((NGB, B, HID), _F32),
            pltpu.VMEM((NSB, B, W1R), _F32),
            pltpu.VMEM((L,), _F32),
            pltpu.SemaphoreType.DMA((NGB,)),
            pltpu.SemaphoreType.DMA((NSB,)),
            pltpu.VMEM_SHARED((NP, W1R), _F32),
        ],
        compiler_params=sc_params,
    )
    sc2 = pl.kernel(
        _sc2_body,
        out_type=[
            jax.ShapeDtypeStruct((NC, NP, W2R), _F32),
        ],
        mesh=mesh,
        scratch_types=[
            pltpu.VMEM((NCH, B), _I32),
            pltpu.VMEM((NCH, B), _I32),
            pltpu.VMEM((NP,), _F32),
            pltpu.VMEM((NP,), _F32),
            pltpu.VMEM((NGB, B, CP), _F32),
            pltpu.VMEM((NSB, B, W2R), _F32),
            pltpu.VMEM((L,), _F32),
            pltpu.SemaphoreType.DMA((NGB,)),
            pltpu.SemaphoreType.DMA((NSB,)),
            pltpu.VMEM_SHARED((NP, W2R), _F32),
        ],
        compiler_params=sc_params,
    )
    return tc1, tc2, tc3, sc1, sc2


@jax.jit
def kernel(x, edges, W1, att_src1, att_dst1, bias1,
           W2, att_src2, att_dst2, bias2):
    _TC1, _TC2, _TC3, _SC1, _SC2 = _make_kernels()
    loop = jnp.arange(N, dtype=_I32)
    P = E2P - (edges.shape[1] + N)
    pad = jnp.arange(P, dtype=_I32)
    src = jnp.concatenate([edges[0].astype(_I32), loop, pad % N])
    dst = jnp.concatenate([edges[1].astype(_I32), loop, N + pad % (NP - N)])
    src3 = src.reshape(NW, NCH, B)
    dst3 = dst.reshape(NW, NCH, B)

    eye = jnp.eye(HEADS, dtype=_F32)
    amap_s = (att_src1[:, None, :, None] * eye[:, None, None, :]) \
        .reshape(HEADS, DH, HEADS).reshape(HID, HEADS)
    amap_d = (att_dst1[:, None, :, None] * eye[:, None, None, :]) \
        .reshape(HEADS, DH, HEADS).reshape(HID, HEADS)

    xp = jnp.pad(x, ((0, NP - N), (0, 0)))

    e8 = (jnp.eye(HEADS, dtype=_F32)[:, :, None] *
          jnp.ones((DH,), _F32)).reshape(HEADS, HID)

    h1, asd, add_, m1 = _TC1(xp, W1, amap_s, amap_d)
    part1, = _SC1(src3, dst3, asd, add_, h1, m1)
    h2p, as2, ad2, m2 = _TC2(part1, bias1, W2, att_src2, att_dst2, e8)
    part2, = _SC2(src3, dst3, as2.reshape(NP), ad2.reshape(NP), h2p, m2)
    return _TC3(part2, bias2)


# speedup vs baseline: 2.3919x; 2.3919x over previous
import functools

import jax
import jax.numpy as jnp
from jax import lax
from jax.experimental import pallas as pl
from jax.experimental.pallas import tpu as pltpu
from jax.experimental.pallas import tpu_sc as plsc

N = 10000
FEATS = 128
HID = 64
HEADS = 8
DH = 8
CLASSES = 40
CP = 48

NC = 2
NS = 16
NW = NC * NS
L = 16

W1R = HID + L
W2R = CP + L

NP = 10240
STRIPE = NP // NS
B = 128
NCH = 84
NGB = 3
NSB = 2
C = NCH * B
E2P = NW * C

_F32 = jnp.float32
_I32 = jnp.int32


def _iota16():
    return lax.iota(_I32, L)


def _vperm(v, idx):
    dn = lax.GatherDimensionNumbers(
        offset_dims=(), collapsed_slice_dims=(0,), start_index_map=(0,))
    return lax.gather(v, idx[:, None], dn, (1,),
                      mode=lax.GatherScatterMode.PROMISE_IN_BOUNDS)


def _tc1_body(xp_ref, w1_ref, amap_s_ref, amap_d_ref,
              h1_ref, asd_ref, add_ref, m1_ref):
    h = jnp.dot(xp_ref[...], w1_ref[...], preferred_element_type=_F32)
    h1_ref[...] = h
    a_s = jnp.dot(h, amap_s_ref[...], preferred_element_type=_F32)
    a_d = jnp.dot(h, amap_d_ref[...], preferred_element_type=_F32)
    asd_ref[...] = jnp.concatenate([a_s, a_s], axis=1)
    add_ref[...] = jnp.concatenate([a_d, a_d], axis=1)
    m = jnp.maximum(jnp.max(a_s, axis=0) + jnp.max(a_d, axis=0), 0.0)
    m1_ref[...] = jnp.concatenate([m, m], axis=0)


def _tc2_body(part_ref, b1_ref, w2_ref, as2w_ref, ad2w_ref, e8_ref,
              h2p_ref, as2_ref, ad2_ref, m2_ref):
    den = part_ref[0, :, HID:HID + HEADS] + part_ref[1, :, HID:HID + HEADS]
    agg = part_ref[0, :, :HID] + part_ref[1, :, :HID]
    inv = 1.0 / (den + 1e-16)
    invx = jnp.dot(inv, e8_ref[...], preferred_element_type=_F32)
    h1f = agg * invx + b1_ref[...][None, :]
    rowid = lax.broadcasted_iota(_I32, (NP, 1), 0)
    h1f = jnp.where(rowid < N, h1f, 0.0)
    h2 = jnp.dot(h1f, w2_ref[...], preferred_element_type=_F32)
    h2p_ref[...] = jnp.pad(h2, ((0, 0), (0, CP - CLASSES)))
    a_s = jnp.dot(h2, as2w_ref[...].reshape(CLASSES, 1),
                  preferred_element_type=_F32)
    a_d = jnp.dot(h2, ad2w_ref[...].reshape(CLASSES, 1),
                  preferred_element_type=_F32)
    a_s = jnp.where(rowid < N, a_s, 0.0)
    a_d = jnp.where(rowid < N, a_d, 0.0)
    as2_ref[...] = a_s
    ad2_ref[...] = a_d
    m2 = jnp.maximum(jnp.max(a_s) + jnp.max(a_d), 0.0)
    m2_ref[...] = jnp.full((L,), m2, dtype=_F32)


def _tc3_body(part_ref, b2_ref, y_ref):
    den = part_ref[0, :, CP] + part_ref[1, :, CP]
    agg = part_ref[0, :, :CLASSES] + part_ref[1, :, :CLASSES]
    inv = 1.0 / (den + 1e-16)
    y = agg * inv[:, None] + b2_ref[...][None, :]
    y_ref[...] = y[:N, :]


def _sc1_body(src_hbm, dst_hbm, asd_hbm, add_hbm, h1_hbm, m1_hbm,
              part_out,
              sidx, didx, sbuf, dbuf, hbuf, msgb, mvec,
              gsem, ssem, out_sh):
    c = lax.axis_index("c")
    s = lax.axis_index("s")
    w = c * NS + s

    def _z(i, _):
        for q in range(W1R // L):
            msgb[0, i, pl.ds(q * L, L)] = jnp.zeros((L,), _F32)
        return 0
    lax.fori_loop(0, B, _z, 0)

    for r in range(STRIPE // B):
        row = s * STRIPE + r * B
        pltpu.sync_copy(msgb.at[0], out_sh.at[pl.ds(row, B)])
    plsc.subcore_barrier()

    pltpu.sync_copy(src_hbm.at[w], sidx)
    pltpu.sync_copy(dst_hbm.at[w], didx)
    pltpu.sync_copy(m1_hbm, mvec)

    def _gathers(ch, g):
        pltpu.async_copy(asd_hbm.at[sidx.at[ch]], sbuf.at[g], gsem.at[g])
        pltpu.async_copy(add_hbm.at[didx.at[ch]], dbuf.at[g], gsem.at[g])
        pltpu.async_copy(h1_hbm.at[sidx.at[ch]], hbuf.at[g], gsem.at[g])

    def _gwait(ch, g):
        pltpu.make_async_copy(asd_hbm.at[sidx.at[ch]], sbuf.at[g],
                              gsem.at[g]).wait()
        pltpu.make_async_copy(add_hbm.at[didx.at[ch]], dbuf.at[g],
                              gsem.at[g]).wait()
        pltpu.make_async_copy(h1_hbm.at[sidx.at[ch]], hbuf.at[g],
                              gsem.at[g]).wait()

    def _swait(ch, mb):
        pltpu.make_async_copy(msgb.at[mb], out_sh.at[didx.at[ch]],
                              ssem.at[mb]).wait()

    for g in range(NGB):
        _gathers(g, g)

    m = mvec[...]
    half = lax.shift_right_logical(_iota16(), 3)
    idxq = [half + (2 * q) for q in range(4)]

    def _super(i, _):
        for b in range(6):
            ch = 6 * i + b
            g = b % NGB
            mb = b % NSB
            _gwait(ch, g)

            @pl.when(ch >= NSB)
            def _():
                _swait(ch - NSB, mb)

            @plsc.parallel_loop(0, B, step=1, unroll=4)
            def _edge(e):
                t = sbuf[g, e, :] + dbuf[g, e, :]
                a = jnp.maximum(t, 0.2 * t)
                ex = jnp.exp(a - m)
                msgb[mb, e, pl.ds(HID, L)] = ex
                for q in range(4):
                    wv = _vperm(ex, idxq[q])
                    msgb[mb, e, pl.ds(q * L, L)] = \
                        wv * hbuf[g, e, pl.ds(q * L, L)]
            pltpu.async_copy(msgb.at[mb], out_sh.at[didx.at[ch]],
                             ssem.at[mb], add=True)

            @pl.when(ch + NGB < NCH)
            def _():
                _gathers(ch + NGB, g)
        return 0

    lax.fori_loop(0, NCH // 6, _super, 0)
    for ch in range(NCH - NSB, NCH):
        _swait(ch, ch % NSB)
    plsc.subcore_barrier()

    row = s * STRIPE
    pltpu.sync_copy(out_sh.at[pl.ds(row, STRIPE)],
                    part_out.at[c].at[pl.ds(row, STRIPE)])


def _sc2_body(src_hbm, dst_hbm, as2_hbm, ad2_hbm, h2_hbm, m2_hbm,
              part_out,
              sidx, didx, astab, adtab, hbuf, msgb, mvec,
              gsem, ssem, out_sh):
    c = lax.axis_index("c")
    s = lax.axis_index("s")
    w = c * NS + s

    def _z(i, _):
        for q in range(W2R // L):
            msgb[0, i, pl.ds(q * L, L)] = jnp.zeros((L,), _F32)
        return 0
    lax.fori_loop(0, B, _z, 0)

    for r in range(STRIPE // B):
        row = s * STRIPE + r * B
        pltpu.sync_copy(msgb.at[0], out_sh.at[pl.ds(row, B)])
    plsc.subcore_barrier()

    pltpu.sync_copy(src_hbm.at[w], sidx)
    pltpu.sync_copy(dst_hbm.at[w], didx)
    pltpu.sync_copy(m2_hbm, mvec)
    pltpu.sync_copy(as2_hbm, astab)
    pltpu.sync_copy(ad2_hbm, adtab)

    def _gwait(ch, g):
        pltpu.make_async_copy(h2_hbm.at[sidx.at[ch]], hbuf.at[g],
                              gsem.at[g]).wait()

    def _swait(ch, mb):
        pltpu.make_async_copy(msgb.at[mb], out_sh.at[didx.at[ch]],
                              ssem.at[mb]).wait()

    for g in range(NGB):
        pltpu.async_copy(h2_hbm.at[sidx.at[g]], hbuf.at[g], gsem.at[g])

    m = mvec[...]

    def _super(i, _):
        for b in range(6):
            ch = 6 * i + b
            g = b % NGB
            mb = b % NSB
            _gwait(ch, g)

            @pl.when(ch >= NSB)
            def _():
                _swait(ch - NSB, mb)

            def _grp(gr, _):
                sv = sidx[ch, pl.ds(gr * L, L)]
                dv = didx[ch, pl.ds(gr * L, L)]
                t = (plsc.load_gather(astab, [sv]) +
                     plsc.load_gather(adtab, [dv]))
                a = jnp.maximum(t, 0.2 * t)
                exg = jnp.exp(a - m)

                @plsc.parallel_loop(0, L, step=1, unroll=4)
                def _edge(i2):
                    e = gr * L + i2
                    wv = _vperm(exg, jnp.full((L,), i2, dtype=_I32))
                    msgb[mb, e, pl.ds(CP, L)] = wv
                    for q in range(3):
                        msgb[mb, e, pl.ds(q * L, L)] = \
                            wv * hbuf[g, e, pl.ds(q * L, L)]
                return 0

            lax.fori_loop(0, B // L, _grp, 0)
            pltpu.async_copy(msgb.at[mb], out_sh.at[didx.at[ch]],
                             ssem.at[mb], add=True)

            @pl.when(ch + NGB < NCH)
            def _():
                pltpu.async_copy(h2_hbm.at[sidx.at[ch + NGB]], hbuf.at[g],
                                 gsem.at[g])
        return 0

    lax.fori_loop(0, NCH // 6, _super, 0)
    for ch in range(NCH - NSB, NCH):
        _swait(ch, ch % NSB)
    plsc.subcore_barrier()

    row = s * STRIPE
    pltpu.sync_copy(out_sh.at[pl.ds(row, STRIPE)],
                    part_out.at[c].at[pl.ds(row, STRIPE)])


@functools.lru_cache(maxsize=1)
def _make_kernels():
    tc1 = pl.pallas_call(
        _tc1_body,
        out_shape=[
            jax.ShapeDtypeStruct((NP, HID), _F32),
            jax.ShapeDtypeStruct((NP, 2 * HEADS), _F32),
            jax.ShapeDtypeStruct((NP, 2 * HEADS), _F32),
            jax.ShapeDtypeStruct((L,), _F32),
        ],
    )
    tc2 = pl.pallas_call(
        _tc2_body,
        out_shape=[
            jax.ShapeDtypeStruct((NP, CP), _F32),
            jax.ShapeDtypeStruct((NP, 1), _F32),
            jax.ShapeDtypeStruct((NP, 1), _F32),
            jax.ShapeDtypeStruct((L,), _F32),
        ],
    )
    tc3 = pl.pallas_call(
        _tc3_body,
        out_shape=jax.ShapeDtypeStruct((N, CLASSES), _F32),
    )
    mesh = plsc.VectorSubcoreMesh(
        core_axis_name="c", subcore_axis_name="s",
        num_cores=NC, num_subcores=NS)
    sc_params = pltpu.CompilerParams(use_tc_tiling_on_sc=False,
                                     needs_layout_passes=False)
    sc1 = pl.kernel(
        _sc1_body,
        out_type=[
            jax.ShapeDtypeStruct((NC, NP, W1R), _F32),
        ],
        mesh=mesh,
        scratch_types=[
            pltpu.VMEM((NCH, B), _I32),
            pltpu.VMEM((NCH, B), _I32),
            pltpu.VMEM((NGB, B, 2 * HEADS), _F32),
            pltpu.VMEM((NGB, B, 2 * HEADS), _F32),
            pltpu.VMEM((NGB, B, HID), _F32),
            pltpu.VMEM((NSB, B, W1R), _F32),
            pltpu.VMEM((L,), _F32),
            pltpu.SemaphoreType.DMA((NGB,)),
            pltpu.SemaphoreType.DMA((NSB,)),
            pltpu.VMEM_SHARED((NP, W1R), _F32),
        ],
        compiler_params=sc_params,
    )
    sc2 = pl.kernel(
        _sc2_body,
        out_type=[
            jax.ShapeDtypeStruct((NC, NP, W2R), _F32),
        ],
        mesh=mesh,
        scratch_types=[
            pltpu.VMEM((NCH, B), _I32),
            pltpu.VMEM((NCH, B), _I32),
            pltpu.VMEM((NP,), _F32),
            pltpu.VMEM((NP,), _F32),
            pltpu.VMEM((NGB, B, CP), _F32),
            pltpu.VMEM((NSB, B, W2R), _F32),
            pltpu.VMEM((L,), _F32),
            pltpu.SemaphoreType.DMA((NGB,)),
            pltpu.SemaphoreType.DMA((NSB,)),
            pltpu.VMEM_SHARED((NP, W2R), _F32),
        ],
        compiler_params=sc_params,
    )
    return tc1, tc2, tc3, sc1, sc2


@jax.jit
def kernel(x, edges, W1, att_src1, att_dst1, bias1,
           W2, att_src2, att_dst2, bias2):
    _TC1, _TC2, _TC3, _SC1, _SC2 = _make_kernels()
    loop = jnp.arange(N, dtype=_I32)
    P = E2P - (edges.shape[1] + N)
    pad = jnp.arange(P, dtype=_I32)
    src = jnp.concatenate([edges[0].astype(_I32), loop, pad % N])
    dst = jnp.concatenate([edges[1].astype(_I32), loop, N + pad % (NP - N)])
    src3 = src.reshape(NW, NCH, B)
    dst3 = dst.reshape(NW, NCH, B)

    eye = jnp.eye(HEADS, dtype=_F32)
    amap_s = (att_src1[:, None, :, None] * eye[:, None, None, :]) \
        .reshape(HEADS, DH, HEADS).reshape(HID, HEADS)
    amap_d = (att_dst1[:, None, :, None] * eye[:, None, None, :]) \
        .reshape(HEADS, DH, HEADS).reshape(HID, HEADS)

    xp = jnp.pad(x, ((0, NP - N), (0, 0)))

    e8 = (jnp.eye(HEADS, dtype=_F32)[:, :, None] *
          jnp.ones((DH,), _F32)).reshape(HEADS, HID)

    h1, asd, add_, m1 = _TC1(xp, W1, amap_s, amap_d)
    part1, = _SC1(src3, dst3, asd, add_, h1, m1)
    h2p, as2, ad2, m2 = _TC2(part1, bias1, W2, att_src2, att_dst2, e8)
    part2, = _SC2(src3, dst3, as2.reshape(NP), ad2.reshape(NP), h2p, m2)
    return _TC3(part2, bias2)


# speedup vs baseline: 2.5157x; 1.0518x over previous
import functools

import numpy as np

import jax
import jax.numpy as jnp
from jax import lax
from jax.experimental import pallas as pl
from jax.experimental.pallas import tpu as pltpu
from jax.experimental.pallas import tpu_sc as plsc

N = 10000
FEATS = 128
HID = 64
HEADS = 8
DH = 8
CLASSES = 40
CP = 48

NC = 2
NS = 16
NW = NC * NS
L = 16

W1R = HID + L
W2R = CP + L

NP = 10240
STRIPE = NP // NS
B = 128
NCH = 84
NGB = 3
NSB = 2
C = NCH * B
E2P = NW * C
E = 320000
E_ROWS = E // B
FULLW = E_ROWS // NCH
SPLITR = E_ROWS - FULLW * NCH
TAILR = (E2P - E) // B
TAIL_OFF = NCH - SPLITR

_P2 = E2P - E - N
_TAIL_SRC = np.concatenate([np.arange(N), np.arange(_P2) % N]) \
    .astype(np.int32).reshape(TAILR, B)
_TAIL_DST = np.concatenate([np.arange(N), N + np.arange(_P2) % (NP - N)]) \
    .astype(np.int32).reshape(TAILR, B)

_F32 = jnp.float32
_I32 = jnp.int32


def _iota16():
    return lax.iota(_I32, L)


def _vperm(v, idx):
    dn = lax.GatherDimensionNumbers(
        offset_dims=(), collapsed_slice_dims=(0,), start_index_map=(0,))
    return lax.gather(v, idx[:, None], dn, (1,),
                      mode=lax.GatherScatterMode.PROMISE_IN_BOUNDS)


def _tc1_body(xp_ref, w1_ref, amap_s_ref, amap_d_ref,
              h1_ref, asd_ref, add_ref, m1_ref):
    h = jnp.dot(xp_ref[...], w1_ref[...], preferred_element_type=_F32)
    h1_ref[...] = h
    a_s = jnp.dot(h, amap_s_ref[...], preferred_element_type=_F32)
    a_d = jnp.dot(h, amap_d_ref[...], preferred_element_type=_F32)
    asd_ref[...] = jnp.concatenate([a_s, a_s], axis=1)
    add_ref[...] = jnp.concatenate([a_d, a_d], axis=1)
    m = jnp.maximum(jnp.max(a_s, axis=0) + jnp.max(a_d, axis=0), 0.0)
    m1_ref[...] = jnp.concatenate([m, m], axis=0)


def _tc2_body(part_ref, b1_ref, w2_ref, as2w_ref, ad2w_ref, e8_ref,
              h2p_ref, as2_ref, ad2_ref, m2_ref):
    den = part_ref[0, :, HID:HID + HEADS] + part_ref[1, :, HID:HID + HEADS]
    agg = part_ref[0, :, :HID] + part_ref[1, :, :HID]
    inv = 1.0 / (den + 1e-16)
    invx = jnp.dot(inv, e8_ref[...], preferred_element_type=_F32)
    h1f = agg * invx + b1_ref[...][None, :]
    rowid = lax.broadcasted_iota(_I32, (NP, 1), 0)
    h1f = jnp.where(rowid < N, h1f, 0.0)
    h2 = jnp.dot(h1f, w2_ref[...], preferred_element_type=_F32)
    h2p_ref[...] = jnp.pad(h2, ((0, 0), (0, CP - CLASSES)))
    a_s = jnp.dot(h2, as2w_ref[...].reshape(CLASSES, 1),
                  preferred_element_type=_F32)
    a_d = jnp.dot(h2, ad2w_ref[...].reshape(CLASSES, 1),
                  preferred_element_type=_F32)
    a_s = jnp.where(rowid < N, a_s, 0.0)
    a_d = jnp.where(rowid < N, a_d, 0.0)
    as2_ref[...] = a_s
    ad2_ref[...] = a_d
    m2 = jnp.maximum(jnp.max(a_s) + jnp.max(a_d), 0.0)
    m2_ref[...] = jnp.full((L,), m2, dtype=_F32)


def _tc3_body(part_ref, b2_ref, y_ref):
    den = part_ref[0, :, CP] + part_ref[1, :, CP]
    agg = part_ref[0, :, :CLASSES] + part_ref[1, :, :CLASSES]
    inv = 1.0 / (den + 1e-16)
    y = agg * inv[:, None] + b2_ref[...][None, :]
    y_ref[...] = y[:N, :]


def _stage_idx(w, src_hbm, dst_hbm, tsrc_hbm, tdst_hbm, sidx, didx):
    @pl.when(w < FULLW)
    def _():
        pltpu.sync_copy(src_hbm.at[pl.ds(w * NCH, NCH)], sidx)
        pltpu.sync_copy(dst_hbm.at[pl.ds(w * NCH, NCH)], didx)

    @pl.when(w == FULLW)
    def _():
        pltpu.sync_copy(src_hbm.at[pl.ds(FULLW * NCH, SPLITR)],
                        sidx.at[pl.ds(0, SPLITR)])
        pltpu.sync_copy(tsrc_hbm.at[pl.ds(0, TAIL_OFF)],
                        sidx.at[pl.ds(SPLITR, TAIL_OFF)])
        pltpu.sync_copy(dst_hbm.at[pl.ds(FULLW * NCH, SPLITR)],
                        didx.at[pl.ds(0, SPLITR)])
        pltpu.sync_copy(tdst_hbm.at[pl.ds(0, TAIL_OFF)],
                        didx.at[pl.ds(SPLITR, TAIL_OFF)])

    @pl.when(w > FULLW)
    def _():
        off = TAIL_OFF + (w - FULLW - 1) * NCH
        pltpu.sync_copy(tsrc_hbm.at[pl.ds(off, NCH)], sidx)
        pltpu.sync_copy(tdst_hbm.at[pl.ds(off, NCH)], didx)


def _sc1_body(src_hbm, dst_hbm, tsrc_hbm, tdst_hbm,
              asd_hbm, add_hbm, h1_hbm, m1_hbm,
              part_out,
              sidx, didx, sbuf, dbuf, hbuf, msgb, mvec,
              gsem, ssem, out_sh):
    c = lax.axis_index("c")
    s = lax.axis_index("s")
    w = c * NS + s

    @plsc.parallel_loop(0, B, step=1, unroll=4)
    def _z(i):
        for q in range(W1R // L):
            msgb[0, i, pl.ds(q * L, L)] = jnp.zeros((L,), _F32)

    for r in range(STRIPE // B):
        row = s * STRIPE + r * B
        pltpu.sync_copy(msgb.at[0], out_sh.at[pl.ds(row, B)])
    plsc.subcore_barrier()

    _stage_idx(w, src_hbm, dst_hbm, tsrc_hbm, tdst_hbm, sidx, didx)
    pltpu.sync_copy(m1_hbm, mvec)

    def _gathers(ch, g):
        pltpu.async_copy(asd_hbm.at[sidx.at[ch]], sbuf.at[g], gsem.at[g])
        pltpu.async_copy(add_hbm.at[didx.at[ch]], dbuf.at[g], gsem.at[g])
        pltpu.async_copy(h1_hbm.at[sidx.at[ch]], hbuf.at[g], gsem.at[g])

    def _gwait(ch, g):
        pltpu.make_async_copy(asd_hbm.at[sidx.at[ch]], sbuf.at[g],
                              gsem.at[g]).wait()
        pltpu.make_async_copy(add_hbm.at[didx.at[ch]], dbuf.at[g],
                              gsem.at[g]).wait()
        pltpu.make_async_copy(h1_hbm.at[sidx.at[ch]], hbuf.at[g],
                              gsem.at[g]).wait()

    def _swait(ch, mb):
        pltpu.make_async_copy(msgb.at[mb], out_sh.at[didx.at[ch]],
                              ssem.at[mb]).wait()

    for g in range(NGB):
        _gathers(g, g)

    m = mvec[...]
    half = lax.shift_right_logical(_iota16(), 3)
    idxq = [half + (2 * q) for q in range(4)]

    def _super(i, _):
        for b in range(6):
            ch = 6 * i + b
            g = b % NGB
            mb = b % NSB
            _gwait(ch, g)

            @pl.when(ch >= NSB)
            def _():
                _swait(ch - NSB, mb)

            @plsc.parallel_loop(0, B, step=1, unroll=4)
            def _edge(e):
                t = sbuf[g, e, :] + dbuf[g, e, :]
                a = jnp.maximum(t, 0.2 * t)
                ex = jnp.exp(a - m)
                msgb[mb, e, pl.ds(HID, L)] = ex
                for q in range(4):
                    wv = _vperm(ex, idxq[q])
                    msgb[mb, e, pl.ds(q * L, L)] = \
                        wv * hbuf[g, e, pl.ds(q * L, L)]
            pltpu.async_copy(msgb.at[mb], out_sh.at[didx.at[ch]],
                             ssem.at[mb], add=True)

            @pl.when(ch + NGB < NCH)
            def _():
                _gathers(ch + NGB, g)
        return 0

    lax.fori_loop(0, NCH // 6, _super, 0)
    for ch in range(NCH - NSB, NCH):
        _swait(ch, ch % NSB)
    plsc.subcore_barrier()

    row = s * STRIPE
    pltpu.sync_copy(out_sh.at[pl.ds(row, STRIPE)],
                    part_out.at[c].at[pl.ds(row, STRIPE)])


def _sc2_body(src_hbm, dst_hbm, tsrc_hbm, tdst_hbm,
              as2_hbm, ad2_hbm, h2_hbm, m2_hbm,
              part_out,
              sidx, didx, astab, adtab, hbuf, msgb, mvec,
              gsem, ssem, out_sh):
    c = lax.axis_index("c")
    s = lax.axis_index("s")
    w = c * NS + s

    @plsc.parallel_loop(0, B, step=1, unroll=4)
    def _z(i):
        for q in range(W2R // L):
            msgb[0, i, pl.ds(q * L, L)] = jnp.zeros((L,), _F32)

    for r in range(STRIPE // B):
        row = s * STRIPE + r * B
        pltpu.sync_copy(msgb.at[0], out_sh.at[pl.ds(row, B)])
    plsc.subcore_barrier()

    _stage_idx(w, src_hbm, dst_hbm, tsrc_hbm, tdst_hbm, sidx, didx)
    pltpu.sync_copy(m2_hbm, mvec)
    pltpu.sync_copy(as2_hbm, astab)
    pltpu.sync_copy(ad2_hbm, adtab)

    def _gwait(ch, g):
        pltpu.make_async_copy(h2_hbm.at[sidx.at[ch]], hbuf.at[g],
                              gsem.at[g]).wait()

    def _swait(ch, mb):
        pltpu.make_async_copy(msgb.at[mb], out_sh.at[didx.at[ch]],
                              ssem.at[mb]).wait()

    for g in range(NGB):
        pltpu.async_copy(h2_hbm.at[sidx.at[g]], hbuf.at[g], gsem.at[g])

    m = mvec[...]

    def _super(i, _):
        for b in range(6):
            ch = 6 * i + b
            g = b % NGB
            mb = b % NSB
            _gwait(ch, g)

            @pl.when(ch >= NSB)
            def _():
                _swait(ch - NSB, mb)

            def _grp(gr, _):
                sv = sidx[ch, pl.ds(gr * L, L)]
                dv = didx[ch, pl.ds(gr * L, L)]
                t = (plsc.load_gather(astab, [sv]) +
                     plsc.load_gather(adtab, [dv]))
                a = jnp.maximum(t, 0.2 * t)
                exg = jnp.exp(a - m)

                @plsc.parallel_loop(0, L, step=1, unroll=8)
                def _edge(i2):
                    e = gr * L + i2
                    wv = _vperm(exg, jnp.full((L,), i2, dtype=_I32))
                    msgb[mb, e, pl.ds(CP, L)] = wv
                    for q in range(3):
                        msgb[mb, e, pl.ds(q * L, L)] = \
                            wv * hbuf[g, e, pl.ds(q * L, L)]
                return 0

            lax.fori_loop(0, B // L, _grp, 0)
            pltpu.async_copy(msgb.at[mb], out_sh.at[didx.at[ch]],
                             ssem.at[mb], add=True)

            @pl.when(ch + NGB < NCH)
            def _():
                pltpu.async_copy(h2_hbm.at[sidx.at[ch + NGB]], hbuf.at[g],
                                 gsem.at[g])
        return 0

    lax.fori_loop(0, NCH // 6, _super, 0)
    for ch in range(NCH - NSB, NCH):
        _swait(ch, ch % NSB)
    plsc.subcore_barrier()

    row = s * STRIPE
    pltpu.sync_copy(out_sh.at[pl.ds(row, STRIPE)],
                    part_out.at[c].at[pl.ds(row, STRIPE)])


@functools.lru_cache(maxsize=1)
def _make_kernels():
    tc1 = pl.pallas_call(
        _tc1_body,
        out_shape=[
            jax.ShapeDtypeStruct((NP, HID), _F32),
            jax.ShapeDtypeStruct((NP, 2 * HEADS), _F32),
            jax.ShapeDtypeStruct((NP, 2 * HEADS), _F32),
            jax.ShapeDtypeStruct((L,), _F32),
        ],
    )
    tc2 = pl.pallas_call(
        _tc2_body,
        out_shape=[
            jax.ShapeDtypeStruct((NP, CP), _F32),
            jax.ShapeDtypeStruct((NP, 1), _F32),
            jax.ShapeDtypeStruct((NP, 1), _F32),
            jax.ShapeDtypeStruct((L,), _F32),
        ],
    )
    tc3 = pl.pallas_call(
        _tc3_body,
        out_shape=jax.ShapeDtypeStruct((N, CLASSES), _F32),
    )
    mesh = plsc.VectorSubcoreMesh(
        core_axis_name="c", subcore_axis_name="s",
        num_cores=NC, num_subcores=NS)
    sc_params = pltpu.CompilerParams(use_tc_tiling_on_sc=False,
                                     needs_layout_passes=False)
    sc1 = pl.kernel(
        _sc1_body,
        out_type=[
            jax.ShapeDtypeStruct((NC, NP, W1R), _F32),
        ],
        mesh=mesh,
        scratch_types=[
            pltpu.VMEM((NCH, B), _I32),
            pltpu.VMEM((NCH, B), _I32),
            pltpu.VMEM((NGB, B, 2 * HEADS), _F32),
            pltpu.VMEM((NGB, B, 2 * HEADS), _F32),
            pltpu.VMEM((NGB, B, HID), _F32),
            pltpu.VMEM((NSB, B, W1R), _F32),
            pltpu.VMEM((L,), _F32),
            pltpu.SemaphoreType.DMA((NGB,)),
            pltpu.SemaphoreType.DMA((NSB,)),
            pltpu.VMEM_SHARED((NP, W1R), _F32),
        ],
        compiler_params=sc_params,
    )
    sc2 = pl.kernel(
        _sc2_body,
        out_type=[
            jax.ShapeDtypeStruct((NC, NP, W2R), _F32),
        ],
        mesh=mesh,
        scratch_types=[
            pltpu.VMEM((NCH, B), _I32),
            pltpu.VMEM((NCH, B), _I32),
            pltpu.VMEM((NP,), _F32),
            pltpu.VMEM((NP,), _F32),
            pltpu.VMEM((NGB, B, CP), _F32),
            pltpu.VMEM((NSB, B, W2R), _F32),
            pltpu.VMEM((L,), _F32),
            pltpu.SemaphoreType.DMA((NGB,)),
            pltpu.SemaphoreType.DMA((NSB,)),
            pltpu.VMEM_SHARED((NP, W2R), _F32),
        ],
        compiler_params=sc_params,
    )
    return tc1, tc2, tc3, sc1, sc2


@jax.jit
def kernel(x, edges, W1, att_src1, att_dst1, bias1,
           W2, att_src2, att_dst2, bias2):
    _TC1, _TC2, _TC3, _SC1, _SC2 = _make_kernels()
    esrc = edges[0].astype(_I32).reshape(E_ROWS, B)
    edst = edges[1].astype(_I32).reshape(E_ROWS, B)
    tsrc = jnp.asarray(_TAIL_SRC)
    tdst = jnp.asarray(_TAIL_DST)

    eye = jnp.eye(HEADS, dtype=_F32)
    amap_s = (att_src1[:, None, :, None] * eye[:, None, None, :]) \
        .reshape(HEADS, DH, HEADS).reshape(HID, HEADS)
    amap_d = (att_dst1[:, None, :, None] * eye[:, None, None, :]) \
        .reshape(HEADS, DH, HEADS).reshape(HID, HEADS)

    xp = jnp.pad(x, ((0, NP - N), (0, 0)))

    e8 = (jnp.eye(HEADS, dtype=_F32)[:, :, None] *
          jnp.ones((DH,), _F32)).reshape(HEADS, HID)

    h1, asd, add_, m1 = _TC1(xp, W1, amap_s, amap_d)
    part1, = _SC1(esrc, edst, tsrc, tdst, asd, add_, h1, m1)
    h2p, as2, ad2, m2 = _TC2(part1, bias1, W2, att_src2, att_dst2, e8)
    part2, = _SC2(esrc, edst, tsrc, tdst,
                  as2.reshape(NP), ad2.reshape(NP), h2p, m2)
    return _TC3(part2, bias2)


# speedup vs baseline: 2.5289x; 1.0052x over previous
import functools

import numpy as np

import jax
import jax.numpy as jnp
from jax import lax
from jax.experimental import pallas as pl
from jax.experimental.pallas import tpu as pltpu
from jax.experimental.pallas import tpu_sc as plsc

N = 10000
FEATS = 128
HID = 64
HEADS = 8
DH = 8
CLASSES = 40
CP = 48

NC = 2
NS = 16
NW = NC * NS
L = 16

W1R = HID + L
W2R = CP + L

NP = 10240
STRIPE = NP // NS
B = 128
NCH = 84
NGB = 3
NSB = 2
C = NCH * B
E2P = NW * C
E = 320000
E_ROWS = E // B
FULLW = E_ROWS // NCH
SPLITR = E_ROWS - FULLW * NCH
TAILR = (E2P - E) // B
TAIL_OFF = NCH - SPLITR

_P2 = E2P - E - N
_TAIL_SRC = np.concatenate([np.arange(N), np.arange(_P2) % N]) \
    .astype(np.int32).reshape(TAILR, B)
_TAIL_DST = np.concatenate([np.arange(N), N + np.arange(_P2) % (NP - N)]) \
    .astype(np.int32).reshape(TAILR, B)

_F32 = jnp.float32
_I32 = jnp.int32


def _iota16():
    return lax.iota(_I32, L)


def _vperm(v, idx):
    dn = lax.GatherDimensionNumbers(
        offset_dims=(), collapsed_slice_dims=(0,), start_index_map=(0,))
    return lax.gather(v, idx[:, None], dn, (1,),
                      mode=lax.GatherScatterMode.PROMISE_IN_BOUNDS)


def _tc1_body(xp_ref, w1_ref, amap_s_ref, amap_d_ref,
              h1_ref, asd_ref, add_ref, m1_ref):
    h = jnp.dot(xp_ref[...], w1_ref[...], preferred_element_type=_F32)
    h1_ref[...] = h
    a_s = jnp.dot(h, amap_s_ref[...], preferred_element_type=_F32)
    a_d = jnp.dot(h, amap_d_ref[...], preferred_element_type=_F32)
    asd_ref[...] = jnp.concatenate([a_s, a_s], axis=1)
    add_ref[...] = jnp.concatenate([a_d, a_d], axis=1)
    m = jnp.maximum(jnp.max(a_s, axis=0) + jnp.max(a_d, axis=0), 0.0)
    m1_ref[...] = jnp.concatenate([m, m], axis=0)


def _tc2_body(part_ref, b1_ref, w2p_ref, aw_ref, e8_ref,
              h2p_ref, asad_ref, m2_ref):
    den = part_ref[0, :, HID:HID + HEADS] + part_ref[1, :, HID:HID + HEADS]
    agg = part_ref[0, :, :HID] + part_ref[1, :, :HID]
    inv = 1.0 / (den + 1e-16)
    invx = jnp.dot(inv, e8_ref[...], preferred_element_type=_F32)
    h1f = agg * invx + b1_ref[...][None, :]
    rowid = lax.broadcasted_iota(_I32, (NP, 1), 0)
    h1f = jnp.where(rowid < N, h1f, 0.0)
    h2 = jnp.dot(h1f, w2p_ref[...], preferred_element_type=_F32)
    h2p_ref[...] = h2
    asad = jnp.dot(h2, aw_ref[...], preferred_element_type=_F32)
    asad = jnp.where(rowid < N, asad, 0.0)
    asad_ref[...] = asad
    m2 = jnp.maximum(jnp.max(asad[:, 0]) + jnp.max(asad[:, 1]), 0.0)
    m2_ref[...] = jnp.full((L,), m2, dtype=_F32)


def _tc3_body(part_ref, b2_ref, y_ref):
    den = part_ref[0, :, CP] + part_ref[1, :, CP]
    agg = part_ref[0, :, :CLASSES] + part_ref[1, :, :CLASSES]
    inv = 1.0 / (den + 1e-16)
    y = agg * inv[:, None] + b2_ref[...][None, :]
    y_ref[...] = y[:N, :]


def _stage_idx(w, src_hbm, dst_hbm, tsrc_hbm, tdst_hbm, sidx, didx):
    @pl.when(w < FULLW)
    def _():
        pltpu.sync_copy(src_hbm.at[pl.ds(w * NCH, NCH)], sidx)
        pltpu.sync_copy(dst_hbm.at[pl.ds(w * NCH, NCH)], didx)

    @pl.when(w == FULLW)
    def _():
        pltpu.sync_copy(src_hbm.at[pl.ds(FULLW * NCH, SPLITR)],
                        sidx.at[pl.ds(0, SPLITR)])
        pltpu.sync_copy(tsrc_hbm.at[pl.ds(0, TAIL_OFF)],
                        sidx.at[pl.ds(SPLITR, TAIL_OFF)])
        pltpu.sync_copy(dst_hbm.at[pl.ds(FULLW * NCH, SPLITR)],
                        didx.at[pl.ds(0, SPLITR)])
        pltpu.sync_copy(tdst_hbm.at[pl.ds(0, TAIL_OFF)],
                        didx.at[pl.ds(SPLITR, TAIL_OFF)])

    @pl.when(w > FULLW)
    def _():
        off = TAIL_OFF + (w - FULLW - 1) * NCH
        pltpu.sync_copy(tsrc_hbm.at[pl.ds(off, NCH)], sidx)
        pltpu.sync_copy(tdst_hbm.at[pl.ds(off, NCH)], didx)


def _sc1_body(src_hbm, dst_hbm, tsrc_hbm, tdst_hbm,
              asd_hbm, add_hbm, h1_hbm, m1_hbm,
              part_out,
              sidx, didx, sbuf, dbuf, hbuf, msgb, mvec,
              gsem, ssem, out_sh):
    c = lax.axis_index("c")
    s = lax.axis_index("s")
    w = c * NS + s

    @plsc.parallel_loop(0, B, step=1, unroll=4)
    def _z(i):
        for q in range(W1R // L):
            msgb[0, i, pl.ds(q * L, L)] = jnp.zeros((L,), _F32)

    for r in range(STRIPE // B):
        row = s * STRIPE + r * B
        pltpu.sync_copy(msgb.at[0], out_sh.at[pl.ds(row, B)])
    plsc.subcore_barrier()

    _stage_idx(w, src_hbm, dst_hbm, tsrc_hbm, tdst_hbm, sidx, didx)
    pltpu.sync_copy(m1_hbm, mvec)

    def _gathers(ch, g):
        pltpu.async_copy(asd_hbm.at[sidx.at[ch]], sbuf.at[g], gsem.at[g])
        pltpu.async_copy(add_hbm.at[didx.at[ch]], dbuf.at[g], gsem.at[g])
        pltpu.async_copy(h1_hbm.at[sidx.at[ch]], hbuf.at[g], gsem.at[g])

    def _gwait(ch, g):
        pltpu.make_async_copy(asd_hbm.at[sidx.at[ch]], sbuf.at[g],
                              gsem.at[g]).wait()
        pltpu.make_async_copy(add_hbm.at[didx.at[ch]], dbuf.at[g],
                              gsem.at[g]).wait()
        pltpu.make_async_copy(h1_hbm.at[sidx.at[ch]], hbuf.at[g],
                              gsem.at[g]).wait()

    def _swait(ch, mb):
        pltpu.make_async_copy(msgb.at[mb], out_sh.at[didx.at[ch]],
                              ssem.at[mb]).wait()

    for g in range(NGB):
        _gathers(g, g)

    m = mvec[...]
    half = lax.shift_right_logical(_iota16(), 3)
    idxq = [half + (2 * q) for q in range(4)]

    def _super(i, _):
        for b in range(6):
            ch = 6 * i + b
            g = b % NGB
            mb = b % NSB
            _gwait(ch, g)

            @pl.when(ch >= NSB)
            def _():
                _swait(ch - NSB, mb)

            @plsc.parallel_loop(0, B, step=1, unroll=4)
            def _edge(e):
                t = sbuf[g, e, :] + dbuf[g, e, :]
                a = jnp.maximum(t, 0.2 * t)
                ex = jnp.exp(a - m)
                msgb[mb, e, pl.ds(HID, L)] = ex
                for q in range(4):
                    wv = _vperm(ex, idxq[q])
                    msgb[mb, e, pl.ds(q * L, L)] = \
                        wv * hbuf[g, e, pl.ds(q * L, L)]
            pltpu.async_copy(msgb.at[mb], out_sh.at[didx.at[ch]],
                             ssem.at[mb], add=True)

            @pl.when(ch + NGB < NCH)
            def _():
                _gathers(ch + NGB, g)
        return 0

    lax.fori_loop(0, NCH // 6, _super, 0)
    for ch in range(NCH - NSB, NCH):
        _swait(ch, ch % NSB)
    plsc.subcore_barrier()

    row = s * STRIPE
    pltpu.sync_copy(out_sh.at[pl.ds(row, STRIPE)],
                    part_out.at[c].at[pl.ds(row, STRIPE)])


def _sc2_body(src_hbm, dst_hbm, tsrc_hbm, tdst_hbm,
              asad_hbm, h2_hbm, m2_hbm,
              part_out,
              sidx, didx, abtab, hbuf, msgb, mvec,
              gsem, ssem, out_sh):
    c = lax.axis_index("c")
    s = lax.axis_index("s")
    w = c * NS + s

    @plsc.parallel_loop(0, B, step=1, unroll=4)
    def _z(i):
        for q in range(W2R // L):
            msgb[0, i, pl.ds(q * L, L)] = jnp.zeros((L,), _F32)

    for r in range(STRIPE // B):
        row = s * STRIPE + r * B
        pltpu.sync_copy(msgb.at[0], out_sh.at[pl.ds(row, B)])
    plsc.subcore_barrier()

    _stage_idx(w, src_hbm, dst_hbm, tsrc_hbm, tdst_hbm, sidx, didx)
    pltpu.sync_copy(m2_hbm, mvec)
    pltpu.sync_copy(asad_hbm, abtab)

    def _gwait(ch, g):
        pltpu.make_async_copy(h2_hbm.at[sidx.at[ch]], hbuf.at[g],
                              gsem.at[g]).wait()

    def _swait(ch, mb):
        pltpu.make_async_copy(msgb.at[mb], out_sh.at[didx.at[ch]],
                              ssem.at[mb]).wait()

    for g in range(NGB):
        pltpu.async_copy(h2_hbm.at[sidx.at[g]], hbuf.at[g], gsem.at[g])

    m = mvec[...]

    def _super(i, _):
        for b in range(6):
            ch = 6 * i + b
            g = b % NGB
            mb = b % NSB
            _gwait(ch, g)

            @pl.when(ch >= NSB)
            def _():
                _swait(ch - NSB, mb)

            def _grp(gr, _):
                sv = sidx[ch, pl.ds(gr * L, L)]
                dv = didx[ch, pl.ds(gr * L, L)]
                t = (plsc.load_gather(abtab, [sv + sv]) +
                     plsc.load_gather(abtab, [dv + dv + 1]))
                a = jnp.maximum(t, 0.2 * t)
                exg = jnp.exp(a - m)

                @plsc.parallel_loop(0, L, step=1, unroll=8)
                def _edge(i2):
                    e = gr * L + i2
                    wv = _vperm(exg, jnp.full((L,), i2, dtype=_I32))
                    msgb[mb, e, pl.ds(CP, L)] = wv
                    for q in range(3):
                        msgb[mb, e, pl.ds(q * L, L)] = \
                            wv * hbuf[g, e, pl.ds(q * L, L)]
                return 0

            lax.fori_loop(0, B // L, _grp, 0)
            pltpu.async_copy(msgb.at[mb], out_sh.at[didx.at[ch]],
                             ssem.at[mb], add=True)

            @pl.when(ch + NGB < NCH)
            def _():
                pltpu.async_copy(h2_hbm.at[sidx.at[ch + NGB]], hbuf.at[g],
                                 gsem.at[g])
        return 0

    lax.fori_loop(0, NCH // 6, _super, 0)
    for ch in range(NCH - NSB, NCH):
        _swait(ch, ch % NSB)
    plsc.subcore_barrier()

    row = s * STRIPE
    pltpu.sync_copy(out_sh.at[pl.ds(row, STRIPE)],
                    part_out.at[c].at[pl.ds(row, STRIPE)])


@functools.lru_cache(maxsize=1)
def _make_kernels():
    tc1 = pl.pallas_call(
        _tc1_body,
        out_shape=[
            jax.ShapeDtypeStruct((NP, HID), _F32),
            jax.ShapeDtypeStruct((NP, 2 * HEADS), _F32),
            jax.ShapeDtypeStruct((NP, 2 * HEADS), _F32),
            jax.ShapeDtypeStruct((L,), _F32),
        ],
    )
    tc2 = pl.pallas_call(
        _tc2_body,
        out_shape=[
            jax.ShapeDtypeStruct((NP, CP), _F32),
            jax.ShapeDtypeStruct((NP, 2), _F32),
            jax.ShapeDtypeStruct((L,), _F32),
        ],
    )
    tc3 = pl.pallas_call(
        _tc3_body,
        out_shape=jax.ShapeDtypeStruct((N, CLASSES), _F32),
    )
    mesh = plsc.VectorSubcoreMesh(
        core_axis_name="c", subcore_axis_name="s",
        num_cores=NC, num_subcores=NS)
    sc_params = pltpu.CompilerParams(use_tc_tiling_on_sc=False,
                                     needs_layout_passes=False)
    sc1 = pl.kernel(
        _sc1_body,
        out_type=[
            jax.ShapeDtypeStruct((NC, NP, W1R), _F32),
        ],
        mesh=mesh,
        scratch_types=[
            pltpu.VMEM((NCH, B), _I32),
            pltpu.VMEM((NCH, B), _I32),
            pltpu.VMEM((NGB, B, 2 * HEADS), _F32),
            pltpu.VMEM((NGB, B, 2 * HEADS), _F32),
            pltpu.VMEM((NGB, B, HID), _F32),
            pltpu.VMEM((NSB, B, W1R), _F32),
            pltpu.VMEM((L,), _F32),
            pltpu.SemaphoreType.DMA((NGB,)),
            pltpu.SemaphoreType.DMA((NSB,)),
            pltpu.VMEM_SHARED((NP, W1R), _F32),
        ],
        compiler_params=sc_params,
    )
    sc2 = pl.kernel(
        _sc2_body,
        out_type=[
            jax.ShapeDtypeStruct((NC, NP, W2R), _F32),
        ],
        mesh=mesh,
        scratch_types=[
            pltpu.VMEM((NCH, B), _I32),
            pltpu.VMEM((NCH, B), _I32),
            pltpu.VMEM((2 * NP,), _F32),
            pltpu.VMEM((NGB, B, CP), _F32),
            pltpu.VMEM((NSB, B, W2R), _F32),
            pltpu.VMEM((L,), _F32),
            pltpu.SemaphoreType.DMA((NGB,)),
            pltpu.SemaphoreType.DMA((NSB,)),
            pltpu.VMEM_SHARED((NP, W2R), _F32),
        ],
        compiler_params=sc_params,
    )
    return tc1, tc2, tc3, sc1, sc2


@jax.jit
def kernel(x, edges, W1, att_src1, att_dst1, bias1,
           W2, att_src2, att_dst2, bias2):
    _TC1, _TC2, _TC3, _SC1, _SC2 = _make_kernels()
    esrc = edges[0].astype(_I32).reshape(E_ROWS, B)
    edst = edges[1].astype(_I32).reshape(E_ROWS, B)
    tsrc = jnp.asarray(_TAIL_SRC)
    tdst = jnp.asarray(_TAIL_DST)

    eye = jnp.eye(HEADS, dtype=_F32)
    amap_s = (att_src1[:, None, :, None] * eye[:, None, None, :]) \
        .reshape(HEADS, DH, HEADS).reshape(HID, HEADS)
    amap_d = (att_dst1[:, None, :, None] * eye[:, None, None, :]) \
        .reshape(HEADS, DH, HEADS).reshape(HID, HEADS)

    xp = jnp.pad(x, ((0, NP - N), (0, 0)))

    e8 = (jnp.eye(HEADS, dtype=_F32)[:, :, None] *
          jnp.ones((DH,), _F32)).reshape(HEADS, HID)

    h1, asd, add_, m1 = _TC1(xp, W1, amap_s, amap_d)
    part1, = _SC1(esrc, edst, tsrc, tdst, asd, add_, h1, m1)
    w2p = jnp.pad(W2, ((0, 0), (0, CP - CLASSES)))
    aw = jnp.pad(jnp.concatenate([att_src2, att_dst2], axis=0).T,
                 ((0, CP - CLASSES), (0, 0)))
    h2p, asad, m2 = _TC2(part1, bias1, w2p, aw, e8)
    part2, = _SC2(esrc, edst, tsrc, tdst, asad.reshape(2 * NP), h2p, m2)
    return _TC3(part2, bias2)


# speedup vs baseline: 2.6225x; 1.0370x over previous
import functools

import numpy as np

import jax
import jax.numpy as jnp
from jax import lax
from jax.experimental import pallas as pl
from jax.experimental.pallas import tpu as pltpu
from jax.experimental.pallas import tpu_sc as plsc

N = 10000
FEATS = 128
HID = 64
HEADS = 8
DH = 8
CLASSES = 40
CP = 48

NC = 2
NS = 16
NW = NC * NS
L = 16

W1R = HID + L
W2R = CP + L

NP = 10240
STRIPE = NP // NS
B = 128
NCH = 84
NGB = 3
NSB = 2
C = NCH * B
E2P = NW * C
E = 320000
E_ROWS = E // B
FULLW = E_ROWS // NCH
SPLITR = E_ROWS - FULLW * NCH
TAILR = (E2P - E) // B
TAIL_OFF = NCH - SPLITR

_P2 = E2P - E - N
_TAIL_SRC = np.concatenate([np.arange(N), np.arange(_P2) % N]) \
    .astype(np.int32).reshape(TAILR, B)
_TAIL_DST = np.concatenate([np.arange(N), N + np.arange(_P2) % (NP - N)]) \
    .astype(np.int32).reshape(TAILR, B)

_F32 = jnp.float32
_I32 = jnp.int32


def _iota16():
    return lax.iota(_I32, L)


def _vperm(v, idx):
    dn = lax.GatherDimensionNumbers(
        offset_dims=(), collapsed_slice_dims=(0,), start_index_map=(0,))
    return lax.gather(v, idx[:, None], dn, (1,),
                      mode=lax.GatherScatterMode.PROMISE_IN_BOUNDS)


def _tc1_body(xp_ref, w1_ref, amap_s_ref, amap_d_ref,
              h1_ref, asd_ref, add_ref, m1_ref):
    h = jnp.dot(xp_ref[...], w1_ref[...], preferred_element_type=_F32)
    h1_ref[...] = h
    a_s = jnp.dot(h, amap_s_ref[...], preferred_element_type=_F32)
    a_d = jnp.dot(h, amap_d_ref[...], preferred_element_type=_F32)
    asd_ref[...] = jnp.concatenate([a_s, a_s], axis=1)
    add_ref[...] = jnp.concatenate([a_d, a_d], axis=1)
    m = jnp.maximum(jnp.max(a_s, axis=0) + jnp.max(a_d, axis=0), 0.0)
    m1_ref[...] = jnp.concatenate([m, m], axis=0)


def _tc2_body(part_ref, b1_ref, w2p_ref, aw_ref, e8_ref,
              h2p_ref, asad_ref, m2_ref):
    den = part_ref[0, :, HID:HID + HEADS] + part_ref[1, :, HID:HID + HEADS]
    agg = part_ref[0, :, :HID] + part_ref[1, :, :HID]
    inv = 1.0 / (den + 1e-16)
    invx = jnp.dot(inv, e8_ref[...], preferred_element_type=_F32)
    h1f = agg * invx + b1_ref[...][None, :]
    rowid = lax.broadcasted_iota(_I32, (NP, 1), 0)
    h1f = jnp.where(rowid < N, h1f, 0.0)
    h2 = jnp.dot(h1f, w2p_ref[...], preferred_element_type=_F32)
    h2p_ref[...] = h2
    asad = jnp.dot(h2, aw_ref[...], preferred_element_type=_F32)
    asad = jnp.where(rowid < N, asad, 0.0)
    asad_ref[...] = asad
    m2 = jnp.maximum(jnp.max(asad[:, 0]) + jnp.max(asad[:, 1]), 0.0)
    m2_ref[...] = jnp.full((L,), m2, dtype=_F32)


def _tc3_body(part_ref, b2_ref, y_ref):
    den = part_ref[0, :, CP] + part_ref[1, :, CP]
    agg = part_ref[0, :, :CLASSES] + part_ref[1, :, :CLASSES]
    inv = 1.0 / (den + 1e-16)
    y = agg * inv[:, None] + b2_ref[...][None, :]
    y_ref[...] = y[:N, :]


def _stage_idx(w, e_hbm, tsrc_hbm, tdst_hbm, sidx, didx):
    src_hbm = e_hbm.at[0]
    dst_hbm = e_hbm.at[1]

    @pl.when(w < FULLW)
    def _():
        pltpu.sync_copy(src_hbm.at[pl.ds(w * NCH, NCH)], sidx)
        pltpu.sync_copy(dst_hbm.at[pl.ds(w * NCH, NCH)], didx)

    @pl.when(w == FULLW)
    def _():
        pltpu.sync_copy(src_hbm.at[pl.ds(FULLW * NCH, SPLITR)],
                        sidx.at[pl.ds(0, SPLITR)])
        pltpu.sync_copy(tsrc_hbm.at[pl.ds(0, TAIL_OFF)],
                        sidx.at[pl.ds(SPLITR, TAIL_OFF)])
        pltpu.sync_copy(dst_hbm.at[pl.ds(FULLW * NCH, SPLITR)],
                        didx.at[pl.ds(0, SPLITR)])
        pltpu.sync_copy(tdst_hbm.at[pl.ds(0, TAIL_OFF)],
                        didx.at[pl.ds(SPLITR, TAIL_OFF)])

    @pl.when(w > FULLW)
    def _():
        off = TAIL_OFF + (w - FULLW - 1) * NCH
        pltpu.sync_copy(tsrc_hbm.at[pl.ds(off, NCH)], sidx)
        pltpu.sync_copy(tdst_hbm.at[pl.ds(off, NCH)], didx)


def _sc1_body(e_hbm, tsrc_hbm, tdst_hbm,
              asd_hbm, add_hbm, h1_hbm, m1_hbm,
              part_out,
              sidx, didx, sbuf, dbuf, hbuf, msgb, mvec,
              gsem, ssem, out_sh):
    c = lax.axis_index("c")
    s = lax.axis_index("s")
    w = c * NS + s

    @plsc.parallel_loop(0, B, step=1, unroll=4)
    def _z(i):
        for q in range(W1R // L):
            msgb[0, i, pl.ds(q * L, L)] = jnp.zeros((L,), _F32)

    for r in range(STRIPE // B):
        row = s * STRIPE + r * B
        pltpu.sync_copy(msgb.at[0], out_sh.at[pl.ds(row, B)])
    plsc.subcore_barrier()

    _stage_idx(w, e_hbm, tsrc_hbm, tdst_hbm, sidx, didx)
    pltpu.sync_copy(m1_hbm, mvec)

    def _gathers(ch, g):
        pltpu.async_copy(asd_hbm.at[sidx.at[ch]], sbuf.at[g], gsem.at[g])
        pltpu.async_copy(add_hbm.at[didx.at[ch]], dbuf.at[g], gsem.at[g])
        pltpu.async_copy(h1_hbm.at[sidx.at[ch]], hbuf.at[g], gsem.at[g])

    def _gwait(ch, g):
        pltpu.make_async_copy(asd_hbm.at[sidx.at[ch]], sbuf.at[g],
                              gsem.at[g]).wait()
        pltpu.make_async_copy(add_hbm.at[didx.at[ch]], dbuf.at[g],
                              gsem.at[g]).wait()
        pltpu.make_async_copy(h1_hbm.at[sidx.at[ch]], hbuf.at[g],
                              gsem.at[g]).wait()

    def _swait(ch, mb):
        pltpu.make_async_copy(msgb.at[mb], out_sh.at[didx.at[ch]],
                              ssem.at[mb]).wait()

    for g in range(NGB):
        _gathers(g, g)

    m = mvec[...]
    half = lax.shift_right_logical(_iota16(), 3)
    idxq = [half + (2 * q) for q in range(4)]

    def _super(i, _):
        for b in range(6):
            ch = 6 * i + b
            g = b % NGB
            mb = b % NSB
            _gwait(ch, g)

            @pl.when(ch >= NSB)
            def _():
                _swait(ch - NSB, mb)

            @plsc.parallel_loop(0, B, step=1, unroll=4)
            def _edge(e):
                t = sbuf[g, e, :] + dbuf[g, e, :]
                a = jnp.maximum(t, 0.2 * t)
                ex = jnp.exp(a - m)
                msgb[mb, e, pl.ds(HID, L)] = ex
                for q in range(4):
                    wv = _vperm(ex, idxq[q])
                    msgb[mb, e, pl.ds(q * L, L)] = \
                        wv * hbuf[g, e, pl.ds(q * L, L)]
            pltpu.async_copy(msgb.at[mb], out_sh.at[didx.at[ch]],
                             ssem.at[mb], add=True)

            @pl.when(ch + NGB < NCH)
            def _():
                _gathers(ch + NGB, g)
        return 0

    lax.fori_loop(0, NCH // 6, _super, 0)
    for ch in range(NCH - NSB, NCH):
        _swait(ch, ch % NSB)
    plsc.subcore_barrier()

    row = s * STRIPE
    pltpu.sync_copy(out_sh.at[pl.ds(row, STRIPE)],
                    part_out.at[c].at[pl.ds(row, STRIPE)])


def _sc2_body(e_hbm, tsrc_hbm, tdst_hbm,
              asad_hbm, h2_hbm, m2_hbm,
              part_out,
              sidx, didx, abtab, hbuf, msgb, mvec,
              gsem, ssem, out_sh):
    c = lax.axis_index("c")
    s = lax.axis_index("s")
    w = c * NS + s

    @plsc.parallel_loop(0, B, step=1, unroll=4)
    def _z(i):
        for q in range(W2R // L):
            msgb[0, i, pl.ds(q * L, L)] = jnp.zeros((L,), _F32)

    for r in range(STRIPE // B):
        row = s * STRIPE + r * B
        pltpu.sync_copy(msgb.at[0], out_sh.at[pl.ds(row, B)])
    plsc.subcore_barrier()

    _stage_idx(w, e_hbm, tsrc_hbm, tdst_hbm, sidx, didx)
    pltpu.sync_copy(m2_hbm, mvec)
    pltpu.sync_copy(asad_hbm, abtab)

    def _gwait(ch, g):
        pltpu.make_async_copy(h2_hbm.at[sidx.at[ch]], hbuf.at[g],
                              gsem.at[g]).wait()

    def _swait(ch, mb):
        pltpu.make_async_copy(msgb.at[mb], out_sh.at[didx.at[ch]],
                              ssem.at[mb]).wait()

    for g in range(NGB):
        pltpu.async_copy(h2_hbm.at[sidx.at[g]], hbuf.at[g], gsem.at[g])

    m = mvec[...]

    def _super(i, _):
        for b in range(6):
            ch = 6 * i + b
            g = b % NGB
            mb = b % NSB
            _gwait(ch, g)

            @pl.when(ch >= NSB)
            def _():
                _swait(ch - NSB, mb)

            def _grp(gr, _):
                sv = sidx[ch, pl.ds(gr * L, L)]
                dv = didx[ch, pl.ds(gr * L, L)]
                t = (plsc.load_gather(abtab, [sv + sv]) +
                     plsc.load_gather(abtab, [dv + dv + 1]))
                a = jnp.maximum(t, 0.2 * t)
                exg = jnp.exp(a - m)

                @plsc.parallel_loop(0, L, step=1, unroll=8)
                def _edge(i2):
                    e = gr * L + i2
                    wv = _vperm(exg, jnp.full((L,), i2, dtype=_I32))
                    msgb[mb, e, pl.ds(CP, L)] = wv
                    for q in range(3):
                        msgb[mb, e, pl.ds(q * L, L)] = \
                            wv * hbuf[g, e, pl.ds(q * L, L)]
                return 0

            lax.fori_loop(0, B // L, _grp, 0)
            pltpu.async_copy(msgb.at[mb], out_sh.at[didx.at[ch]],
                             ssem.at[mb], add=True)

            @pl.when(ch + NGB < NCH)
            def _():
                pltpu.async_copy(h2_hbm.at[sidx.at[ch + NGB]], hbuf.at[g],
                                 gsem.at[g])
        return 0

    lax.fori_loop(0, NCH // 6, _super, 0)
    for ch in range(NCH - NSB, NCH):
        _swait(ch, ch % NSB)
    plsc.subcore_barrier()

    row = s * STRIPE
    pltpu.sync_copy(out_sh.at[pl.ds(row, STRIPE)],
                    part_out.at[c].at[pl.ds(row, STRIPE)])


@functools.lru_cache(maxsize=1)
def _make_kernels():
    tc1 = pl.pallas_call(
        _tc1_body,
        out_shape=[
            jax.ShapeDtypeStruct((NP, HID), _F32),
            jax.ShapeDtypeStruct((NP, 2 * HEADS), _F32),
            jax.ShapeDtypeStruct((NP, 2 * HEADS), _F32),
            jax.ShapeDtypeStruct((L,), _F32),
        ],
    )
    tc2 = pl.pallas_call(
        _tc2_body,
        out_shape=[
            jax.ShapeDtypeStruct((NP, CP), _F32),
            jax.ShapeDtypeStruct((NP, 2), _F32),
            jax.ShapeDtypeStruct((L,), _F32),
        ],
    )
    tc3 = pl.pallas_call(
        _tc3_body,
        out_shape=jax.ShapeDtypeStruct((N, CLASSES), _F32),
    )
    mesh = plsc.VectorSubcoreMesh(
        core_axis_name="c", subcore_axis_name="s",
        num_cores=NC, num_subcores=NS)
    sc_params = pltpu.CompilerParams(use_tc_tiling_on_sc=False,
                                     needs_layout_passes=False)
    sc1 = pl.kernel(
        _sc1_body,
        out_type=[
            jax.ShapeDtypeStruct((NC, NP, W1R), _F32),
        ],
        mesh=mesh,
        scratch_types=[
            pltpu.VMEM((NCH, B), _I32),
            pltpu.VMEM((NCH, B), _I32),
            pltpu.VMEM((NGB, B, 2 * HEADS), _F32),
            pltpu.VMEM((NGB, B, 2 * HEADS), _F32),
            pltpu.VMEM((NGB, B, HID), _F32),
            pltpu.VMEM((NSB, B, W1R), _F32),
            pltpu.VMEM((L,), _F32),
            pltpu.SemaphoreType.DMA((NGB,)),
            pltpu.SemaphoreType.DMA((NSB,)),
            pltpu.VMEM_SHARED((NP, W1R), _F32),
        ],
        compiler_params=sc_params,
    )
    sc2 = pl.kernel(
        _sc2_body,
        out_type=[
            jax.ShapeDtypeStruct((NC, NP, W2R), _F32),
        ],
        mesh=mesh,
        scratch_types=[
            pltpu.VMEM((NCH, B), _I32),
            pltpu.VMEM((NCH, B), _I32),
            pltpu.VMEM((2 * NP,), _F32),
            pltpu.VMEM((NGB, B, CP), _F32),
            pltpu.VMEM((NSB, B, W2R), _F32),
            pltpu.VMEM((L,), _F32),
            pltpu.SemaphoreType.DMA((NGB,)),
            pltpu.SemaphoreType.DMA((NSB,)),
            pltpu.VMEM_SHARED((NP, W2R), _F32),
        ],
        compiler_params=sc_params,
    )
    return tc1, tc2, tc3, sc1, sc2


@jax.jit
def kernel(x, edges, W1, att_src1, att_dst1, bias1,
           W2, att_src2, att_dst2, bias2):
    _TC1, _TC2, _TC3, _SC1, _SC2 = _make_kernels()
    e2d = edges.astype(_I32).reshape(2, E_ROWS, B)
    tsrc = jnp.asarray(_TAIL_SRC)
    tdst = jnp.asarray(_TAIL_DST)

    eye = jnp.eye(HEADS, dtype=_F32)
    amap_s = (att_src1[:, None, :, None] * eye[:, None, None, :]) \
        .reshape(HEADS, DH, HEADS).reshape(HID, HEADS)
    amap_d = (att_dst1[:, None, :, None] * eye[:, None, None, :]) \
        .reshape(HEADS, DH, HEADS).reshape(HID, HEADS)

    xp = jnp.pad(x, ((0, NP - N), (0, 0)))

    e8 = (jnp.eye(HEADS, dtype=_F32)[:, :, None] *
          jnp.ones((DH,), _F32)).reshape(HEADS, HID)

    h1, asd, add_, m1 = _TC1(xp, W1, amap_s, amap_d)
    part1, = _SC1(e2d, tsrc, tdst, asd, add_, h1, m1)
    w2p = jnp.pad(W2, ((0, 0), (0, CP - CLASSES)))
    aw = jnp.pad(jnp.concatenate([att_src2, att_dst2], axis=0).T,
                 ((0, CP - CLASSES), (0, 0)))
    h2p, asad, m2 = _TC2(part1, bias1, w2p, aw, e8)
    part2, = _SC2(e2d, tsrc, tdst, asad.reshape(2 * NP), h2p, m2)
    return _TC3(part2, bias2)
